# Initial kernel scaffold; baseline (speedup 1.0000x reference)
import jax
import jax.numpy as jnp
from jax.experimental import pallas as pl


def kernel(x, edge_index, edge_attr, globalFeats, isTrain, W_rel1, b_rel1, W_root1, W_rel2, b_rel2, W_root2, Wg1, bg1, Wg2, bg2, Wg3, bg3, Wo1, bo1, Wo2, bo2):
    raise NotImplementedError("write your pallas kernel here")


# speedup vs baseline: 4.2940x; 4.2940x over previous
import functools

import jax
import jax.numpy as jnp
from jax import lax
from jax.experimental import pallas as pl
from jax.experimental.pallas import tpu as pltpu
from jax.experimental.pallas import tpu_sc as plsc

N = 27648
E = 442368
B = 512
NPG = 54
NC = 2
NS = 16
CH = 128
ZR = 64
RPT = N // NS

_f32 = jnp.float32
_i32 = jnp.int32


def _tc1_body(x_ref, wlo_ref, whi_ref, wroot_ref, b1_ref,
              xlo_ref, xhi_ref, xroot_ref):
    x = x_ref[...]
    xlo_ref[...] = jnp.dot(x, wlo_ref[...], preferred_element_type=_f32)
    xhi_ref[...] = jnp.dot(x, whi_ref[...], preferred_element_type=_f32)
    xroot_ref[...] = (jnp.dot(x, wroot_ref[...], preferred_element_type=_f32)
                      + b1_ref[...])


def _tc1(x, wlo, whi, wroot, b1):
    blk = 1024
    grid = (N // blk,)
    return pl.pallas_call(
        _tc1_body,
        grid=grid,
        in_specs=[
            pl.BlockSpec((blk, 128), lambda i: (i, 0)),
            pl.BlockSpec((128, 64), lambda i: (0, 0)),
            pl.BlockSpec((128, 64), lambda i: (0, 0)),
            pl.BlockSpec((128, 128), lambda i: (0, 0)),
            pl.BlockSpec((1, 128), lambda i: (0, 0)),
        ],
        out_specs=[
            pl.BlockSpec((blk, 64), lambda i: (i, 0)),
            pl.BlockSpec((blk, 64), lambda i: (i, 0)),
            pl.BlockSpec((blk, 128), lambda i: (i, 0)),
        ],
        out_shape=[
            jax.ShapeDtypeStruct((N, 64), _f32),
            jax.ShapeDtypeStruct((N, 64), _f32),
            jax.ShapeDtypeStruct((N, 128), _f32),
        ],
    )(x, wlo, whi, wroot, b1)


def _sc1_body(src_hbm, dst_hbm, w_hbm, xlo_hbm, xhi_hbm, agg_hbm,
              src_v, dst_v, w_v, rows_v, zbuf, acc_sh, sem):
    c = lax.axis_index("c")
    s = lax.axis_index("s")

    def _zrow(r, carry):
        for q in range(4):
            zbuf[r, pl.ds(q * 16, 16)] = jnp.zeros((16,), _f32)
        return carry
    lax.fori_loop(0, ZR, _zrow, 0)

    def _zcp(i, carry):
        pltpu.sync_copy(zbuf, acc_sh.at[pl.ds(s * RPT + i * ZR, ZR)])
        return carry
    lax.fori_loop(0, RPT // ZR, _zcp, 0)
    plsc.subcore_barrier()

    ept = E // NS

    def _edges(tbl_hbm):
        def _chunk(i, carry):
            base = s * ept + i * CH
            pltpu.sync_copy(src_hbm.at[pl.ds(base, CH)], src_v)
            pltpu.sync_copy(w_hbm.at[pl.ds(base, CH)], w_v)
            pltpu.sync_copy(dst_hbm.at[pl.ds(base, CH)], dst_v)
            pltpu.async_copy(tbl_hbm.at[src_v], rows_v, sem).wait()

            def _edge(e, cc):
                wb = plsc.load_gather(w_v, (jnp.full((16,), e, _i32),))
                for q in range(4):
                    sl = pl.ds(q * 16, 16)
                    rows_v[e, sl] = rows_v[e, sl] * wb
                return cc
            lax.fori_loop(0, CH, _edge, 0)
            pltpu.sync_copy(rows_v, acc_sh.at[dst_v], add=True)
            return carry
        lax.fori_loop(0, ept // CH, _chunk, 0)

    @pl.when(c == 0)
    def _():
        _edges(xlo_hbm)

    @pl.when(c == 1)
    def _():
        _edges(xhi_hbm)

    plsc.subcore_barrier()

    def _wb(i, carry):
        r0 = s * RPT + i * ZR
        pltpu.sync_copy(acc_sh.at[pl.ds(r0, ZR)],
                        agg_hbm.at[c, pl.ds(r0, ZR)])
        return carry
    lax.fori_loop(0, RPT // ZR, _wb, 0)


def _sc1(src, dst, w, xlo, xhi):
    f = pl.kernel(
        _sc1_body,
        out_type=jax.ShapeDtypeStruct((2, N, 64), _f32),
        compiler_params=pltpu.CompilerParams(needs_layout_passes=False, use_tc_tiling_on_sc=False),
        mesh=plsc.VectorSubcoreMesh(core_axis_name="c", subcore_axis_name="s"),
        scratch_types=[
            pltpu.VMEM((CH,), _i32),
            pltpu.VMEM((CH,), _i32),
            pltpu.VMEM((CH,), _f32),
            pltpu.VMEM((CH, 64), _f32),
            pltpu.VMEM((ZR, 64), _f32),
            pltpu.VMEM_SHARED((N, 64), _f32),
            pltpu.SemaphoreType.DMA,
        ],
    )
    return f(src, dst, w, xlo, xhi)


def _tc2_body(agg_ref, xroot_ref, w2p_ref, wr2p_ref, b2p_ref,
              y2p_ref, hroot_ref):
    h_lo = jnp.maximum(agg_ref[0] + xroot_ref[:, :64], 0.0)
    h_hi = jnp.maximum(agg_ref[1] + xroot_ref[:, 64:], 0.0)
    y2p_ref[...] = (jnp.dot(h_lo, w2p_ref[:64], preferred_element_type=_f32)
                    + jnp.dot(h_hi, w2p_ref[64:], preferred_element_type=_f32))
    hroot_ref[...] = (jnp.dot(h_lo, wr2p_ref[:64], preferred_element_type=_f32)
                      + jnp.dot(h_hi, wr2p_ref[64:], preferred_element_type=_f32)
                      + b2p_ref[...])


def _tc2(agg, xroot, w2p, wr2p, b2p):
    blk = 1024
    grid = (N // blk,)
    return pl.pallas_call(
        _tc2_body,
        grid=grid,
        in_specs=[
            pl.BlockSpec((2, blk, 64), lambda i: (0, i, 0)),
            pl.BlockSpec((blk, 128), lambda i: (i, 0)),
            pl.BlockSpec((128, 16), lambda i: (0, 0)),
            pl.BlockSpec((128, 16), lambda i: (0, 0)),
            pl.BlockSpec((1, 16), lambda i: (0, 0)),
        ],
        out_specs=[
            pl.BlockSpec((blk, 16), lambda i: (i, 0)),
            pl.BlockSpec((blk, 16), lambda i: (i, 0)),
        ],
        out_shape=[
            jax.ShapeDtypeStruct((N, 16), _f32),
            jax.ShapeDtypeStruct((N, 16), _f32),
        ],
    )(agg, xroot, w2p, wr2p, b2p)


def _sc2_body(src_hbm, dst_hbm, w_hbm, y2_hbm, acc_hbm,
              src_v, dst_v, w_v, rows_v, zbuf, acc_sh, sem):
    c = lax.axis_index("c")
    s = lax.axis_index("s")

    def _zrow(r, carry):
        zbuf[r, :] = jnp.zeros((16,), _f32)
        return carry
    lax.fori_loop(0, ZR, _zrow, 0)

    def _zcp(i, carry):
        pltpu.sync_copy(zbuf, acc_sh.at[pl.ds(s * RPT + i * ZR, ZR)])
        return carry
    lax.fori_loop(0, RPT // ZR, _zcp, 0)
    plsc.subcore_barrier()

    ept = E // (NC * NS)
    wid = s * NC + c

    def _chunk(i, carry):
        base = wid * ept + i * CH
        pltpu.sync_copy(src_hbm.at[pl.ds(base, CH)], src_v)
        pltpu.sync_copy(w_hbm.at[pl.ds(base, CH)], w_v)
        pltpu.sync_copy(dst_hbm.at[pl.ds(base, CH)], dst_v)
        pltpu.async_copy(y2_hbm.at[src_v], rows_v, sem).wait()

        def _edge(e, cc):
            wb = plsc.load_gather(w_v, (jnp.full((16,), e, _i32),))
            rows_v[e, :] = rows_v[e, :] * wb
            return cc
        lax.fori_loop(0, CH, _edge, 0)
        pltpu.sync_copy(rows_v, acc_sh.at[dst_v], add=True)
        return carry
    lax.fori_loop(0, ept // CH, _chunk, 0)
    plsc.subcore_barrier()

    def _wb(i, carry):
        r0 = s * RPT + i * ZR
        pltpu.sync_copy(acc_sh.at[pl.ds(r0, ZR)],
                        acc_hbm.at[c, pl.ds(r0, ZR)])
        return carry
    lax.fori_loop(0, RPT // ZR, _wb, 0)


def _sc2(src, dst, w, y2p):
    f = pl.kernel(
        _sc2_body,
        out_type=jax.ShapeDtypeStruct((2, N, 16), _f32),
        compiler_params=pltpu.CompilerParams(needs_layout_passes=False, use_tc_tiling_on_sc=False),
        mesh=plsc.VectorSubcoreMesh(core_axis_name="c", subcore_axis_name="s"),
        scratch_types=[
            pltpu.VMEM((CH,), _i32),
            pltpu.VMEM((CH,), _i32),
            pltpu.VMEM((CH,), _f32),
            pltpu.VMEM((CH, 16), _f32),
            pltpu.VMEM((ZR, 16), _f32),
            pltpu.VMEM_SHARED((N, 16), _f32),
            pltpu.SemaphoreType.DMA,
        ],
    )
    return f(src, dst, w, y2p)


def _head_body(accA_ref, accB_ref, hroot_ref, scl_ref, sclg_ref, gf_ref,
               wg1_ref, bg1_ref, wg2_ref, bg2_ref, wg3_ref, bg3_ref,
               wo1a_ref, wo1b_ref, bo1_ref, wo2_ref, bo2_ref, out_ref):
    h2 = jnp.maximum(accA_ref[...] + accB_ref[...] + hroot_ref[...], 0.0)
    h2 = h2 * scl_ref[...]
    g = jnp.maximum(jnp.dot(gf_ref[...], wg1_ref[...],
                            preferred_element_type=_f32) + bg1_ref[...], 0.0)
    g = jnp.maximum(jnp.dot(g, wg2_ref[...],
                            preferred_element_type=_f32) + bg2_ref[...], 0.0)
    g = jnp.maximum(jnp.dot(g, wg3_ref[...],
                            preferred_element_type=_f32) + bg3_ref[...], 0.0)
    g = g * sclg_ref[...]
    o1 = jnp.maximum(
        jnp.dot(h2, wo1a_ref[...], preferred_element_type=_f32)
        + jnp.dot(g, wo1b_ref[...], preferred_element_type=_f32)
        + bo1_ref[...], 0.0)
    o2 = jnp.dot(o1, wo2_ref[...], preferred_element_type=_f32) + bo2_ref[...]
    out_ref[...] = jax.nn.sigmoid(o2)


def _head(accA, accB, hroot, scl, sclg, gf,
          wg1, bg1, wg2, bg2, wg3, bg3, wo1a, wo1b, bo1, wo2, bo2):
    return pl.pallas_call(
        _head_body,
        out_shape=jax.ShapeDtypeStruct((B, 1), _f32),
    )(accA, accB, hroot, scl, sclg, gf,
      wg1, bg1, wg2, bg2, wg3, bg3, wo1a, wo1b, bo1, wo2, bo2)


def kernel(x, edge_index, edge_attr, globalFeats, isTrain,
           W_rel1, b_rel1, W_root1, W_rel2, b_rel2, W_root2,
           Wg1, bg1, Wg2, bg2, Wg3, bg3, Wo1, bo1, Wo2, bo2):
    src = jnp.asarray(edge_index[0], _i32)
    dst = jnp.asarray(edge_index[1], _i32)
    w = jnp.asarray(edge_attr, _f32)

    xlo, xhi, xroot = _tc1(x, W_rel1[:, :64], W_rel1[:, 64:], W_root1,
                           b_rel1.reshape(1, 128))

    agg = _sc1(src, dst, w, xlo, xhi)

    pad = ((0, 0), (0, 12))
    w2p = jnp.pad(W_rel2, pad)
    wr2p = jnp.pad(W_root2, pad)
    b2p = jnp.pad(b_rel2, (0, 12)).reshape(1, 16)
    y2p, hroot2p = _tc2(agg, xroot, w2p, wr2p, b2p)

    acc2 = _sc2(src, dst, w, y2p)

    accA = acc2[0].reshape(B, NPG * 16)
    accB = acc2[1].reshape(B, NPG * 16)
    hroot_r = hroot2p.reshape(B, NPG * 16)

    d_cat = NPG * 4 + 16
    keep = jax.random.bernoulli(jax.random.key(42), 0.8, (B, d_cat))
    scale = jnp.where(jnp.asarray(isTrain),
                      jnp.where(keep, 1.0 / 0.8, 0.0),
                      1.0).astype(_f32)
    scl_emb = jnp.pad(scale[:, :NPG * 4].reshape(B, NPG, 4),
                      ((0, 0), (0, 0), (0, 12))).reshape(B, NPG * 16)
    scl_g = scale[:, NPG * 4:]

    wo1a = jnp.pad(Wo1[:NPG * 4].reshape(NPG, 4, 128),
                   ((0, 0), (0, 12), (0, 0))).reshape(NPG * 16, 128)
    wo1b = Wo1[NPG * 4:]

    return _head(accA, accB, hroot_r, scl_emb, scl_g, globalFeats,
                 Wg1, bg1.reshape(1, 8), Wg2, bg2.reshape(1, 8),
                 Wg3, bg3.reshape(1, 16), wo1a, wo1b, bo1.reshape(1, 128),
                 Wo2, bo2.reshape(1, 1))
